# Initial kernel scaffold; baseline (speedup 1.0000x reference)
import functools

import jax
import jax.numpy as jnp
from jax import lax
from jax.experimental import pallas as pl
from jax.experimental.pallas import tpu as pltpu
from jax.experimental.pallas import tpu_sc as plsc

D = 64
ND = D // 16
NC = 2
NS = 16
NW = NC * NS
C = 512
NG = 4
GSZ = C // NG
EPS = 1e-5
INTERPRET = False


def _invsqrt(v):
    yi = jnp.full((16,), 0x5F3759DF, jnp.int32) - jnp.right_shift(
        plsc.bitcast(v, jnp.int32), 1)
    y = plsc.bitcast(yi, jnp.float32)
    for _ in range(3):
        y = y * (1.5 - 0.5 * v * y * y)
    return y


def _body(x_hbm, seg_hbm, tok_hbm, pos_hbm, segtab_hbm, gam_hbm, bet_hbm,
          out_hbm, pos_v, segtab_v, gam_v, bet_v, idx_v, segc_v, rows_v, sem):
    n_tok, seq_len = x_hbm.shape[0], pos_hbm.shape[0]
    per_w = n_tok // NW
    nch = per_w // C
    wid = lax.axis_index("s") * NC + lax.axis_index("c")
    base = wid * per_w

    pltpu.sync_copy(pos_hbm, pos_v)
    pltpu.sync_copy(segtab_hbm, segtab_v)
    pltpu.sync_copy(gam_hbm, gam_v)
    pltpu.sync_copy(bet_hbm, bet_v)

    s0 = [segtab_v[0, pl.ds(16 * d, 16)] for d in range(ND)]
    sd = [segtab_v[1, pl.ds(16 * d, 16)] - s0[d] for d in range(ND)]
    gam = [gam_v[pl.ds(16 * d, 16)] for d in range(ND)]
    bet = [bet_v[pl.ds(16 * d, 16)] for d in range(ND)]

    def chunk(k, carry):
        off = base + k * C
        pltpu.sync_copy(x_hbm.at[pl.ds(off, C)], idx_v)
        pltpu.sync_copy(seg_hbm.at[pl.ds(off, C)], segc_v)
        cps = [pltpu.async_copy(tok_hbm.at[idx_v.at[pl.ds(g * GSZ, GSZ)]],
                                rows_v.at[pl.ds(g * GSZ, GSZ)], sem)
               for g in range(NG)]
        for cp in cps:
            cp.wait()

        def token(j, carry2):
            l = lax.rem(off + j, seq_len)
            sf = jnp.full((16,), segc_v[j])
            e = []
            for d in range(ND):
                t = rows_v[j, pl.ds(16 * d, 16)]
                p = pos_v[l, pl.ds(16 * d, 16)]
                e.append(t + p + s0[d] + sf * sd[d])
            tot = jnp.sum((e[0] + e[1]) + (e[2] + e[3]))
            totq = jnp.sum((e[0] * e[0] + e[1] * e[1])
                           + (e[2] * e[2] + e[3] * e[3]))
            mean = tot * (1.0 / D)
            var = totq * (1.0 / D) - mean * mean
            mv = jnp.full((16,), mean)
            rstd = _invsqrt(jnp.full((16,), var + EPS))
            for d in range(ND):
                rows_v[j, pl.ds(16 * d, 16)] = (
                    (e[d] - mv) * rstd * gam[d] + bet[d])
            return carry2

        lax.fori_loop(0, C, token, 0)
        pltpu.sync_copy(rows_v, out_hbm.at[pl.ds(off, C)])
        return carry

    lax.fori_loop(0, nch, chunk, 0)


@jax.jit
def kernel(x, seg, tok_table, pos_table, seg_table, gamma, beta):
    b, l = x.shape
    n = b * l
    xf = x.reshape(n).astype(jnp.int32)
    segf = seg.reshape(n).astype(jnp.float32)
    run = pl.kernel(
        _body,
        out_type=jax.ShapeDtypeStruct((n, D), jnp.float32),
        mesh=plsc.VectorSubcoreMesh(core_axis_name="c", subcore_axis_name="s"),
        scratch_types=[
            pltpu.VMEM((l, D), jnp.float32),
            pltpu.VMEM((2, D), jnp.float32),
            pltpu.VMEM((D,), jnp.float32),
            pltpu.VMEM((D,), jnp.float32),
            pltpu.VMEM((C,), jnp.int32),
            pltpu.VMEM((C,), jnp.float32),
            pltpu.VMEM((C, D), jnp.float32),
            pltpu.SemaphoreType.DMA,
        ],
        interpret=INTERPRET,
    )
    out = run(xf, segf, tok_table, pos_table, seg_table, gamma, beta)
    return out.reshape(b, l, D)


# speedup vs baseline: 2.7729x; 2.7729x over previous
import functools

import jax
import jax.numpy as jnp
from jax import lax
from jax.experimental import pallas as pl
from jax.experimental.pallas import tpu as pltpu
from jax.experimental.pallas import tpu_sc as plsc

D = 64
ND = D // 16
NC = 2
NS = 16
NW = NC * NS
C = 512
NG = 4
GSZ = C // NG
EPS = 1e-5
INTERPRET = False


def _invsqrt(v):
    yi = jnp.full((16,), 0x5F3759DF, jnp.int32) - jnp.right_shift(
        plsc.bitcast(v, jnp.int32), 1)
    y = plsc.bitcast(yi, jnp.float32)
    for _ in range(3):
        y = y * (1.5 - 0.5 * v * y * y)
    return y


def _body(x_hbm, seg_hbm, tok_hbm, pos_hbm, segtab_hbm, gam_hbm, bet_hbm,
          out_hbm, pos_v, segtab_v, gam_v, bet_v, idx_v, segc_v, rows_v, sem):
    n_tok, seq_len = x_hbm.shape[0], pos_hbm.shape[0]
    per_w = n_tok // NW
    nch = per_w // C
    wid = lax.axis_index("s") * NC + lax.axis_index("c")
    base = wid * per_w

    pltpu.sync_copy(pos_hbm, pos_v)
    pltpu.sync_copy(segtab_hbm, segtab_v)
    pltpu.sync_copy(gam_hbm, gam_v)
    pltpu.sync_copy(bet_hbm, bet_v)

    s0 = [segtab_v[0, pl.ds(16 * d, 16)] for d in range(ND)]
    sd = [segtab_v[1, pl.ds(16 * d, 16)] - s0[d] for d in range(ND)]
    gam = [gam_v[pl.ds(16 * d, 16)] for d in range(ND)]
    bet = [bet_v[pl.ds(16 * d, 16)] for d in range(ND)]

    def chunk(k, carry):
        off = base + k * C
        pltpu.sync_copy(x_hbm.at[pl.ds(off, C)], idx_v)
        pltpu.sync_copy(seg_hbm.at[pl.ds(off, C)], segc_v.at[pl.ds(0, C)])
        cps = [pltpu.async_copy(tok_hbm.at[idx_v.at[pl.ds(g * GSZ, GSZ)]],
                                rows_v.at[pl.ds(g * GSZ, GSZ)], sem)
               for g in range(NG)]
        for cp in cps:
            cp.wait()

        def token(j, carry2):
            l = lax.rem(off + j, seq_len)
            sf = jnp.full((16,), segc_v[pl.ds(j, 16)][0])
            e = []
            for d in range(ND):
                t = rows_v[j, pl.ds(16 * d, 16)]
                p = pos_v[l, pl.ds(16 * d, 16)]
                e.append(t + p + s0[d] + sf * sd[d])
            tot = jnp.sum((e[0] + e[1]) + (e[2] + e[3]))
            totq = jnp.sum((e[0] * e[0] + e[1] * e[1])
                           + (e[2] * e[2] + e[3] * e[3]))
            mean = tot * (1.0 / D)
            var = totq * (1.0 / D) - mean * mean
            mv = jnp.full((16,), mean)
            rstd = _invsqrt(jnp.full((16,), var + EPS))
            for d in range(ND):
                rows_v[j, pl.ds(16 * d, 16)] = (
                    (e[d] - mv) * rstd * gam[d] + bet[d])
            return carry2

        lax.fori_loop(0, C, token, 0)
        pltpu.sync_copy(rows_v, out_hbm.at[pl.ds(off, C)])
        return carry

    lax.fori_loop(0, nch, chunk, 0)


@jax.jit
def kernel(x, seg, tok_table, pos_table, seg_table, gamma, beta):
    b, l = x.shape
    n = b * l
    xf = x.reshape(n).astype(jnp.int32)
    segf = seg.reshape(n).astype(jnp.float32)
    run = pl.kernel(
        _body,
        out_type=jax.ShapeDtypeStruct((n, D), jnp.float32),
        mesh=plsc.VectorSubcoreMesh(core_axis_name="c", subcore_axis_name="s"),
        scratch_types=[
            pltpu.VMEM((l, D), jnp.float32),
            pltpu.VMEM((2, D), jnp.float32),
            pltpu.VMEM((D,), jnp.float32),
            pltpu.VMEM((D,), jnp.float32),
            pltpu.VMEM((C,), jnp.int32),
            pltpu.VMEM((C + 16,), jnp.float32),
            pltpu.VMEM((C, D), jnp.float32),
            pltpu.SemaphoreType.DMA,
        ],
        compiler_params=pltpu.CompilerParams(
            needs_layout_passes=False, use_tc_tiling_on_sc=False),
        interpret=INTERPRET,
    )
    out = run(xf, segf, tok_table, pos_table, seg_table, gamma, beta)
    return out.reshape(b, l, D)
